# Initial kernel scaffold; baseline (speedup 1.0000x reference)
import jax
import jax.numpy as jnp
from jax.experimental import pallas as pl


def kernel(coords_xyz, grid):
    raise NotImplementedError("write your pallas kernel here")


# speedup vs baseline: 1.0471x; 1.0471x over previous
import functools

import jax
import jax.numpy as jnp
from jax import lax
from jax.experimental import pallas as pl
from jax.experimental.pallas import tpu as pltpu
from jax.experimental.pallas import tpu_sc as plsc

NC = 2
NS = 16
NW = NC * NS
L = 16

GRID = 128
C = 16
N = 262144
V = GRID * GRID * GRID

PTS = N // NW
CHUNK = 256
NCHUNK = PTS // CHUNK
NGRP = CHUNK // L
NIDX = 8 * CHUNK
NDMA = NIDX // 128

_mesh = plsc.VectorSubcoreMesh(core_axis_name="c", subcore_axis_name="s")


@functools.partial(
    pl.kernel,
    out_type=jax.ShapeDtypeStruct((N * C,), jnp.float32),
    mesh=_mesh,
    compiler_params=pltpu.CompilerParams(needs_layout_passes=False,
                                         use_tc_tiling_on_sc=False),
    scratch_types=[
        pltpu.VMEM((3 * CHUNK,), jnp.float32),
        pltpu.VMEM((NIDX,), jnp.int32),
        pltpu.VMEM((NIDX, C), jnp.float32),
        pltpu.VMEM((8 * CHUNK,), jnp.float32),
        pltpu.VMEM((CHUNK * C,), jnp.float32),
        pltpu.SemaphoreType.DMA,
    ],
)
def _sc_sample(coords_t_hbm, table_hbm, out_hbm, cbuf, ibuf, gbuf, wbuf, obuf,
               sem):
    wid = lax.axis_index("s") * NC + lax.axis_index("c")
    base = wid * PTS
    lane = jnp.arange(L, dtype=jnp.int32)

    def chunk_body(t, _):
        start = base + t * CHUNK
        for d in range(3):
            pltpu.sync_copy(coords_t_hbm.at[pl.ds(d * N + start, CHUNK)],
                            cbuf.at[pl.ds(d * CHUNK, CHUNK)])

        def compute_grp(g, _):
            g16 = g * L
            xc = cbuf[pl.ds(g16, L)]
            yc = cbuf[pl.ds(CHUNK + g16, L)]
            zc = cbuf[pl.ds(2 * CHUNK + g16, L)]
            m = ((xc >= -1.0) & (xc <= 1.0) & (yc >= -1.0) & (yc <= 1.0)
                 & (zc >= -1.0) & (zc <= 1.0))
            xc = jnp.where(m, xc, 0.0)
            yc = jnp.where(m, yc, 0.0)
            zc = jnp.where(m, zc, 0.0)
            x = (xc + 1.0) * 0.5 * float(GRID - 1)
            y = (yc + 1.0) * 0.5 * float(GRID - 1)
            z = (zc + 1.0) * 0.5 * float(GRID - 1)
            x0 = x.astype(jnp.int32)
            y0 = y.astype(jnp.int32)
            z0 = z.astype(jnp.int32)
            wx1 = x - x0.astype(jnp.float32)
            wy1 = y - y0.astype(jnp.float32)
            wz1 = z - z0.astype(jnp.float32)
            wx0 = 1.0 - wx1
            wy0 = 1.0 - wy1
            wz0 = 1.0 - wz1
            mf = jnp.where(m, 1.0, 0.0)
            wz0 = wz0 * mf
            wz1 = wz1 * mf
            x1 = jnp.minimum(x0 + 1, GRID - 1)
            y1 = jnp.minimum(y0 + 1, GRID - 1)
            z1 = jnp.minimum(z0 + 1, GRID - 1)

            zy = (
                (z0 * GRID + y0) * GRID,
                (z0 * GRID + y1) * GRID,
                (z1 * GRID + y0) * GRID,
                (z1 * GRID + y1) * GRID,
            )
            wzy = (wz0 * wy0, wz0 * wy1, wz1 * wy0, wz1 * wy1)
            xs = (x0, x1)
            wxs = (wx0, wx1)
            for j in range(8):
                ibuf[pl.ds(j * CHUNK + g16, L)] = zy[j // 2] + xs[j % 2]
                wbuf[pl.ds(j * CHUNK + g16, L)] = wzy[j // 2] * wxs[j % 2]
            return 0

        lax.fori_loop(0, NGRP, compute_grp, 0, unroll=False)

        copies = [
            pltpu.async_copy(table_hbm.at[ibuf.at[pl.ds(b * 128, 128)]],
                             gbuf.at[pl.ds(b * 128, 128)], sem)
            for b in range(NDMA)
        ]
        for cp in copies:
            cp.wait()

        def accum_grp(g, _):
            g16 = g * L
            row0 = g16 + lane
            wv = [wbuf[pl.ds(j * CHUNK + g16, L)] for j in range(8)]
            for c in range(C):
                cc = jnp.full((L,), c, dtype=jnp.int32)
                acc = wv[0] * plsc.load_gather(gbuf, [row0, cc])
                for j in range(1, 8):
                    acc = acc + wv[j] * plsc.load_gather(
                        gbuf, [row0 + j * CHUNK, cc])
                plsc.store_scatter(obuf, [row0 * C + c], acc)
            return 0

        lax.fori_loop(0, NGRP, accum_grp, 0, unroll=False)

        pltpu.sync_copy(obuf, out_hbm.at[pl.ds(start * C, CHUNK * C)])
        return 0

    lax.fori_loop(0, NCHUNK, chunk_body, 0, unroll=False)


def kernel(coords_xyz, grid):
    table = jnp.transpose(grid, (1, 2, 3, 0)).reshape(V, C)
    coords_t = coords_xyz.T.reshape(3 * N)
    return _sc_sample(coords_t, table).reshape(N, C)
